# Initial kernel scaffold; baseline (speedup 1.0000x reference)
import jax
import jax.numpy as jnp
from jax.experimental import pallas as pl


def kernel(boxes, scores):
    raise NotImplementedError("write your pallas kernel here")


# speedup vs baseline: 22.0857x; 22.0857x over previous
import jax
import jax.numpy as jnp
from jax.experimental import pallas as pl
from jax.experimental.pallas import tpu as pltpu
from functools import partial

_N = 5000
_NP = 5120
_ROWS = 8
_COLS = _NP // _ROWS
_NMS_T = 0.3
_SCORE_T = 0.05


def _nms_body(scol_ref, srow_ref, x_ref, o1, o2, o3, o4, o5, rank_ref):
    f32 = jnp.float32

    srow = srow_ref[...]
    gidx_row = jax.lax.broadcasted_iota(jnp.int32, (1, _NP), 1).astype(f32)
    chunk = 640
    for q in range(_NP // chunk):
        sc = scol_ref[q * chunk:(q + 1) * chunk, :]
        gc = jax.lax.broadcasted_iota(jnp.int32, (chunk, 1), 0).astype(f32) + float(q * chunk)
        cmp = (srow > sc) | ((srow == sc) & (gidx_row < gc))
        rank_ref[q * chunk:(q + 1) * chunk, :] = jnp.sum(
            cmp.astype(f32), axis=1, keepdims=True)

    X = x_ref[...]
    rank = rank_ref[...]
    pcol = jax.lax.broadcasted_iota(jnp.int32, (1, _COLS), 1).astype(f32)
    outs = (o1, o2, o3, o4, o5)
    for q in range(_ROWS):
        oh = (rank == (pcol + float(q * _COLS))).astype(f32)
        prod = jax.lax.dot_general(
            X, oh, (((1,), (0,)), ((), ())),
            preferred_element_type=f32,
            precision=jax.lax.Precision.HIGHEST)
        for c in range(5):
            outs[c][q:q + 1, :] = prod[c:c + 1, :]

    x1 = o1[...]
    y1 = o2[...]
    x2 = o3[...]
    y2 = o4[...]
    ss = o5[...]
    area = (x2 - x1) * (y2 - y1)
    pos = (jax.lax.broadcasted_iota(jnp.int32, (_ROWS, _COLS), 0).astype(f32) * float(_COLS)
           + jax.lax.broadcasted_iota(jnp.int32, (_ROWS, _COLS), 1).astype(f32))

    alive0 = (ss > _SCORE_T).astype(f32)
    keep0 = jnp.zeros((_ROWS, _COLS), f32)

    def cond(state):
        alive, _ = state
        return jnp.sum(alive) > 0.0

    def body(state):
        alive, keep = state
        m = jnp.min(jnp.where(alive > 0.0, pos, 2.0 * _NP))
        oh = (pos == m).astype(f32)
        bx1 = jnp.sum(oh * x1)
        by1 = jnp.sum(oh * y1)
        bx2 = jnp.sum(oh * x2)
        by2 = jnp.sum(oh * y2)
        ba = jnp.sum(oh * area)
        w = jnp.maximum(jnp.minimum(x2, bx2) - jnp.maximum(x1, bx1), 0.0)
        h = jnp.maximum(jnp.minimum(y2, by2) - jnp.maximum(y1, by1), 0.0)
        inter = w * h
        union = (ba + area) - inter
        iou = inter / jnp.maximum(union, 1e-9)
        supp = (iou > _NMS_T) & (pos > m)
        alive = jnp.where(supp, 0.0, alive) * (1.0 - oh)
        keep = jnp.maximum(keep, oh)
        return alive, keep

    _, keep = jax.lax.while_loop(cond, body, (alive0, keep0))

    o1[...] = x1 * keep
    o2[...] = y1 * keep
    o3[...] = x2 * keep
    o4[...] = y2 * keep
    o5[...] = ss * keep


@jax.jit
def kernel(boxes, scores):
    n = boxes.shape[0]
    pad = _NP - n
    b = jnp.pad(boxes, ((0, pad), (0, 0)))
    s = jnp.pad(scores, ((0, pad),), constant_values=-1.0)
    scol = s.reshape(_NP, 1)
    srow = s.reshape(1, _NP)
    X = jnp.concatenate([b.T, srow, jnp.zeros((3, _NP), jnp.float32)], axis=0)

    shp = jax.ShapeDtypeStruct((_ROWS, _COLS), jnp.float32)
    o1, o2, o3, o4, o5 = pl.pallas_call(
        _nms_body,
        out_shape=[shp] * 5,
        scratch_shapes=[pltpu.VMEM((_NP, 1), jnp.float32)],
    )(scol, srow, X)

    out = jnp.stack([o1.reshape(_NP), o2.reshape(_NP), o3.reshape(_NP),
                     o4.reshape(_NP), o5.reshape(_NP)], axis=1)
    return out[:n]


# speedup vs baseline: 31.0692x; 1.4068x over previous
import jax
import jax.numpy as jnp
from jax.experimental import pallas as pl
from jax.experimental.pallas import tpu as pltpu
from functools import partial

_N = 5000
_NP = 5120
_ROWS = 8
_COLS = _NP // _ROWS
_NMS_T = 0.3
_SCORE_T = 0.05


def _nms_body(scol_ref, srow_ref, x_ref, o1, o2, o3, o4, o5, rank_ref,
              xst_ref):
    f32 = jnp.float32

    srow = srow_ref[...]
    gidx_row = jax.lax.broadcasted_iota(jnp.int32, (1, _NP), 1).astype(f32)
    chunk = 640
    for q in range(_NP // chunk):
        sc = scol_ref[q * chunk:(q + 1) * chunk, :]
        gc = jax.lax.broadcasted_iota(jnp.int32, (chunk, 1), 0).astype(f32) + float(q * chunk)
        cmp = (srow > sc) | ((srow == sc) & (gidx_row < gc))
        rank_ref[q * chunk:(q + 1) * chunk, :] = jnp.sum(
            cmp.astype(f32), axis=1, keepdims=True)

    X = x_ref[...]
    rank = rank_ref[...]
    pcol = jax.lax.broadcasted_iota(jnp.int32, (1, _COLS), 1).astype(f32)
    outs = (o1, o2, o3, o4, o5)
    for q in range(_ROWS):
        oh = (rank == (pcol + float(q * _COLS))).astype(f32)
        prod = jax.lax.dot_general(
            X, oh, (((1,), (0,)), ((), ())),
            preferred_element_type=f32,
            precision=jax.lax.Precision.HIGHEST)
        for c in range(5):
            outs[c][q:q + 1, :] = prod[c:c + 1, :]
        xst_ref[q * _COLS:(q + 1) * _COLS, :] = jnp.transpose(prod)

    x1 = o1[...]
    y1 = o2[...]
    x2 = o3[...]
    y2 = o4[...]
    ss = o5[...]
    area = (x2 - x1) * (y2 - y1)
    pos = (jax.lax.broadcasted_iota(jnp.int32, (_ROWS, _COLS), 0).astype(f32) * float(_COLS)
           + jax.lax.broadcasted_iota(jnp.int32, (_ROWS, _COLS), 1).astype(f32))

    alive0 = (ss > _SCORE_T).astype(f32)
    keep0 = jnp.zeros((_ROWS, _COLS), f32)
    sent = 2.0 * _NP
    m0 = jnp.min(jnp.where(alive0 > 0.0, pos, sent))

    def cond(state):
        _, _, m = state
        return m < float(_NP)

    def body(state):
        alive, keep, m = state
        sl = xst_ref[pl.ds(m.astype(jnp.int32), 1), :]
        bx1 = sl[0, 0]
        by1 = sl[0, 1]
        bx2 = sl[0, 2]
        by2 = sl[0, 3]
        ba = (bx2 - bx1) * (by2 - by1)
        w = jnp.maximum(jnp.minimum(x2, bx2) - jnp.maximum(x1, bx1), 0.0)
        h = jnp.maximum(jnp.minimum(y2, by2) - jnp.maximum(y1, by1), 0.0)
        inter = w * h
        union = (ba + area) - inter
        iou = inter / jnp.maximum(union, 1e-9)
        supp = (iou > _NMS_T) & (pos > m)
        alive = jnp.where(supp | (pos == m), 0.0, alive)
        keep = jnp.maximum(keep, (pos == m).astype(f32))
        m_next = jnp.min(jnp.where(alive > 0.0, pos, sent))
        return alive, keep, m_next

    _, keep, _ = jax.lax.while_loop(cond, body, (alive0, keep0, m0))

    o1[...] = x1 * keep
    o2[...] = y1 * keep
    o3[...] = x2 * keep
    o4[...] = y2 * keep
    o5[...] = ss * keep


@jax.jit
def kernel(boxes, scores):
    n = boxes.shape[0]
    pad = _NP - n
    b = jnp.pad(boxes, ((0, pad), (0, 0)))
    s = jnp.pad(scores, ((0, pad),), constant_values=-1.0)
    scol = s.reshape(_NP, 1)
    srow = s.reshape(1, _NP)
    X = jnp.concatenate([b.T, srow, jnp.zeros((3, _NP), jnp.float32)], axis=0)

    shp = jax.ShapeDtypeStruct((_ROWS, _COLS), jnp.float32)
    o1, o2, o3, o4, o5 = pl.pallas_call(
        _nms_body,
        out_shape=[shp] * 5,
        scratch_shapes=[pltpu.VMEM((_NP, 1), jnp.float32),
                        pltpu.VMEM((_NP, 8), jnp.float32)],
    )(scol, srow, X)

    out = jnp.stack([o1.reshape(_NP), o2.reshape(_NP), o3.reshape(_NP),
                     o4.reshape(_NP), o5.reshape(_NP)], axis=1)
    return out[:n]


# speedup vs baseline: 33.3032x; 1.0719x over previous
import jax
import jax.numpy as jnp
from jax import lax
from jax.experimental import pallas as pl
from jax.experimental.pallas import tpu as pltpu
from jax.experimental.pallas import tpu_sc as plsc
from functools import partial

_N = 5000
_NP = 5120
_ROWS = 8
_COLS = _NP // _ROWS
_NMS_T = 0.3
_SCORE_T = 0.05

_NC = 2
_NS = 16
_NW = _NC * _NS
_RPW = _NP // _NW
_HALF = _RPW // 2
_W = 16


def _rank_body(scol_ref, srow_ref, rank_ref):
    f32 = jnp.float32
    srow = srow_ref[...]
    gidx_row = lax.broadcasted_iota(jnp.int32, (1, _NP), 1).astype(f32)
    chunk = 640
    for q in range(_NP // chunk):
        sc = scol_ref[q * chunk:(q + 1) * chunk, :]
        gc = lax.broadcasted_iota(jnp.int32, (chunk, 1), 0).astype(f32) \
            + float(q * chunk)
        cmp = (srow > sc) | ((srow == sc) & (gidx_row < gc))
        rank_ref[q * chunk:(q + 1) * chunk, :] = jnp.sum(
            cmp.astype(f32), axis=1, keepdims=True).astype(jnp.int32)


def _sc_permute_body(xt_hbm, rank_hbm, out_hbm, idx_a, idx_b, rows_a, rows_b,
                     sem):
    wid = lax.axis_index("s") * _NC + lax.axis_index("c")
    base = wid * _RPW
    pltpu.sync_copy(rank_hbm.at[pl.ds(base, _HALF)], idx_a)
    pltpu.sync_copy(rank_hbm.at[pl.ds(base + _HALF, _HALF)], idx_b)
    pltpu.sync_copy(xt_hbm.at[pl.ds(base, _HALF), :], rows_a)
    pltpu.sync_copy(xt_hbm.at[pl.ds(base + _HALF, _HALF), :], rows_b)
    pltpu.async_copy(rows_a, out_hbm.at[idx_a], sem).wait()
    pltpu.async_copy(rows_b, out_hbm.at[idx_b], sem).wait()


_sc_permute = pl.kernel(
    _sc_permute_body,
    out_type=jax.ShapeDtypeStruct((_NP, _W), jnp.float32),
    mesh=plsc.VectorSubcoreMesh(core_axis_name="c", subcore_axis_name="s"),
    scratch_types=[
        pltpu.VMEM((_HALF,), jnp.int32),
        pltpu.VMEM((_HALF,), jnp.int32),
        pltpu.VMEM((_HALF, _W), jnp.float32),
        pltpu.VMEM((_HALF, _W), jnp.float32),
        pltpu.SemaphoreType.DMA,
    ],
    compiler_params=pltpu.CompilerParams(use_tc_tiling_on_sc=False),
)


def _nms_body(xst_ref, o1, o2, o3, o4, o5):
    f32 = jnp.float32
    outs = (o1, o2, o3, o4, o5)
    for q in range(_ROWS):
        t = jnp.transpose(xst_ref[q * _COLS:(q + 1) * _COLS, :])
        for c in range(5):
            outs[c][q:q + 1, :] = t[c:c + 1, :]

    x1 = o1[...]
    y1 = o2[...]
    x2 = o3[...]
    y2 = o4[...]
    ss = o5[...]
    area = (x2 - x1) * (y2 - y1)
    pos = (lax.broadcasted_iota(jnp.int32, (_ROWS, _COLS), 0).astype(f32)
           * float(_COLS)
           + lax.broadcasted_iota(jnp.int32, (_ROWS, _COLS), 1).astype(f32))

    alive0 = (ss > _SCORE_T).astype(f32)
    keep0 = jnp.zeros((_ROWS, _COLS), f32)
    sent = 2.0 * _NP
    m0 = jnp.min(jnp.where(alive0 > 0.0, pos, sent))

    def cond(state):
        _, _, m = state
        return m < float(_NP)

    def body(state):
        alive, keep, m = state
        sl = xst_ref[pl.ds(m.astype(jnp.int32), 1), :]
        bx1 = sl[0, 0]
        by1 = sl[0, 1]
        bx2 = sl[0, 2]
        by2 = sl[0, 3]
        ba = (bx2 - bx1) * (by2 - by1)
        w = jnp.maximum(jnp.minimum(x2, bx2) - jnp.maximum(x1, bx1), 0.0)
        h = jnp.maximum(jnp.minimum(y2, by2) - jnp.maximum(y1, by1), 0.0)
        inter = w * h
        union = (ba + area) - inter
        iou = inter / jnp.maximum(union, 1e-9)
        supp = (iou > _NMS_T) & (pos > m)
        alive = jnp.where(supp | (pos == m), 0.0, alive)
        keep = jnp.maximum(keep, (pos == m).astype(f32))
        m_next = jnp.min(jnp.where(alive > 0.0, pos, sent))
        return alive, keep, m_next

    _, keep, _ = jax.lax.while_loop(cond, body, (alive0, keep0, m0))

    o1[...] = x1 * keep
    o2[...] = y1 * keep
    o3[...] = x2 * keep
    o4[...] = y2 * keep
    o5[...] = ss * keep


@jax.jit
def kernel(boxes, scores):
    n = boxes.shape[0]
    pad = _NP - n
    b = jnp.pad(boxes, ((0, pad), (0, 0)))
    s = jnp.pad(scores, ((0, pad),), constant_values=-1.0)
    scol = s.reshape(_NP, 1)
    srow = s.reshape(1, _NP)

    rank = pl.pallas_call(
        _rank_body,
        out_shape=jax.ShapeDtypeStruct((_NP, 1), jnp.int32),
    )(scol, srow)

    xt = jnp.concatenate(
        [b, s[:, None], jnp.zeros((_NP, _W - 5), jnp.float32)], axis=1)
    xst = _sc_permute(xt, rank.reshape(_NP))

    shp = jax.ShapeDtypeStruct((_ROWS, _COLS), jnp.float32)
    o1, o2, o3, o4, o5 = pl.pallas_call(
        _nms_body,
        out_shape=[shp] * 5,
    )(xst)

    out = jnp.stack([o1.reshape(_NP), o2.reshape(_NP), o3.reshape(_NP),
                     o4.reshape(_NP), o5.reshape(_NP)], axis=1)
    return out[:n]


# speedup vs baseline: 39.5184x; 1.1866x over previous
import jax
import jax.numpy as jnp
from jax import lax
from jax.experimental import pallas as pl
from jax.experimental.pallas import tpu as pltpu
from jax.experimental.pallas import tpu_sc as plsc
import functools as _functools

_N = 5000
_NP = 5120
_ROWS = 8
_COLS = _NP // _ROWS
_NMS_T = 0.3
_SCORE_T = 0.05

_NC = 2
_NS = 16
_NW = _NC * _NS
_RPW = _NP // _NW
_HALF = _RPW // 2
_W = 16


def _rank_body(scol_ref, srow_ref, rank_ref):
    f32 = jnp.float32
    srow = srow_ref[...]
    gidx_row = lax.broadcasted_iota(jnp.int32, (1, _NP), 1).astype(f32)
    chunk = 640
    for q in range(_NP // chunk):
        sc = scol_ref[q * chunk:(q + 1) * chunk, :]
        gc = lax.broadcasted_iota(jnp.int32, (chunk, 1), 0).astype(f32) \
            + float(q * chunk)
        cmp = (srow > sc) | ((srow == sc) & (gidx_row < gc))
        rank_ref[q * chunk:(q + 1) * chunk, :] = jnp.sum(
            cmp.astype(f32), axis=1, keepdims=True).astype(jnp.int32)


def _sc_permute_body(xt_hbm, rank_hbm, out_hbm, idx_a, idx_b, rows_a, rows_b,
                     sem):
    wid = lax.axis_index("s") * _NC + lax.axis_index("c")
    base = wid * _RPW
    pltpu.sync_copy(rank_hbm.at[pl.ds(base, _HALF)], idx_a)
    pltpu.sync_copy(rank_hbm.at[pl.ds(base + _HALF, _HALF)], idx_b)
    pltpu.sync_copy(xt_hbm.at[pl.ds(base, _HALF), :], rows_a)
    pltpu.sync_copy(xt_hbm.at[pl.ds(base + _HALF, _HALF), :], rows_b)
    pltpu.async_copy(rows_a, out_hbm.at[idx_a], sem).wait()
    pltpu.async_copy(rows_b, out_hbm.at[idx_b], sem).wait()


@_functools.lru_cache(maxsize=1)
def _get_sc_permute():
    return pl.kernel(
        _sc_permute_body,
        out_type=jax.ShapeDtypeStruct((_NP, _W), jnp.float32),
        mesh=plsc.VectorSubcoreMesh(core_axis_name="c", subcore_axis_name="s"),
        scratch_types=[
            pltpu.VMEM((_HALF,), jnp.int32),
            pltpu.VMEM((_HALF,), jnp.int32),
            pltpu.VMEM((_HALF, _W), jnp.float32),
            pltpu.VMEM((_HALF, _W), jnp.float32),
            pltpu.SemaphoreType.DMA,
        ],
        compiler_params=pltpu.CompilerParams(use_tc_tiling_on_sc=False),
    )


_B = 128


def _nms_body(xst_ref, o1, o2, o3, o4, o5, supp_ref):
    f32 = jnp.float32
    parts = [[] for _ in range(5)]
    for q in range(_ROWS):
        t = jnp.transpose(xst_ref[q * _COLS:(q + 1) * _COLS, :])
        for c in range(5):
            parts[c].append(t[c:c + 1, :])
    x1 = jnp.concatenate(parts[0], axis=1)
    y1 = jnp.concatenate(parts[1], axis=1)
    x2 = jnp.concatenate(parts[2], axis=1)
    y2 = jnp.concatenate(parts[3], axis=1)
    ss = jnp.concatenate(parts[4], axis=1)
    area = (x2 - x1) * (y2 - y1)
    xst = xst_ref[...]

    biota_c = lax.broadcasted_iota(jnp.int32, (_B, 1), 0).astype(f32)
    laneiota = lax.broadcasted_iota(jnp.int32, (1, _B), 1)
    tri = (lax.broadcasted_iota(jnp.int32, (_B, _B), 0)
           < lax.broadcasted_iota(jnp.int32, (_B, _B), 1))

    alive0 = (ss > _SCORE_T).astype(f32)
    keep0 = jnp.zeros((1, _NP), f32)
    cnt0 = jnp.sum(alive0)

    def cond(state):
        return state[2] > 0.0

    def body(state):
        alive, keep, _ = state
        cum = alive
        k = 1
        while k < _NP:
            cum = cum + jnp.pad(cum[:, :-k], ((0, 0), (k, 0)))
            k *= 2
        excl = cum - alive
        self32 = jnp.where((alive > 0.0) & (excl < float(_B)), 1.0, 0.0)
        C = ((excl == biota_c) & (alive > 0.0)).astype(f32)
        blk = jax.lax.dot_general(
            C, xst, (((1,), (0,)), ((), ())),
            preferred_element_type=f32,
            precision=jax.lax.Precision.HIGHEST)
        blkT = jnp.transpose(blk)
        bx1c, by1c, bx2c, by2c = (blk[:, 0:1], blk[:, 1:2],
                                  blk[:, 2:3], blk[:, 3:4])
        bx1r, by1r, bx2r, by2r = (blkT[0:1, :], blkT[1:2, :],
                                  blkT[2:3, :], blkT[3:4, :])
        areac = (bx2c - bx1c) * (by2c - by1c)
        arear = (bx2r - bx1r) * (by2r - by1r)
        wb = jnp.maximum(jnp.minimum(bx2c, bx2r) - jnp.maximum(bx1c, bx1r), 0.0)
        hb = jnp.maximum(jnp.minimum(by2c, by2r) - jnp.maximum(by1c, by1r), 0.0)
        interb = wb * hb
        ioub = interb / jnp.maximum((areac + arear) - interb, 1e-9)
        supp_ref[...] = jnp.where((ioub > _NMS_T) & tri, 1.0, 0.0)

        nm = jnp.sum(self32)
        a0 = (laneiota.astype(f32) < nm).astype(f32)

        def ibody(i, a):
            bit = jnp.sum(a * (laneiota == i).astype(f32))
            srow = supp_ref[pl.ds(i, 1), :]
            return jnp.where(bit > 0.0, a * (1.0 - srow), a)

        a = lax.fori_loop(0, _B, ibody, a0)

        w = jnp.maximum(jnp.minimum(x2, bx2c) - jnp.maximum(x1, bx1c), 0.0)
        h = jnp.maximum(jnp.minimum(y2, by2c) - jnp.maximum(y1, by1c), 0.0)
        inter = w * h
        iou = inter / jnp.maximum((areac + area) - inter, 1e-9)
        P = (iou > _NMS_T).astype(f32)
        deadcnt = jax.lax.dot_general(
            a, P, (((1,), (0,)), ((), ())), preferred_element_type=f32)
        keep = jnp.maximum(keep, jax.lax.dot_general(
            a, C, (((1,), (0,)), ((), ())), preferred_element_type=f32))
        alive = jnp.where(deadcnt > 0.5, 0.0, alive * (1.0 - self32))
        return alive, keep, jnp.sum(alive)

    _, keep, _ = jax.lax.while_loop(cond, body, (alive0, keep0, cnt0))

    o1[...] = x1 * keep
    o2[...] = y1 * keep
    o3[...] = x2 * keep
    o4[...] = y2 * keep
    o5[...] = ss * keep


@jax.jit
def kernel(boxes, scores):
    n = boxes.shape[0]
    pad = _NP - n
    b = jnp.pad(boxes, ((0, pad), (0, 0)))
    s = jnp.pad(scores, ((0, pad),), constant_values=-1.0)
    scol = s.reshape(_NP, 1)
    srow = s.reshape(1, _NP)

    rank = pl.pallas_call(
        _rank_body,
        out_shape=jax.ShapeDtypeStruct((_NP, 1), jnp.int32),
    )(scol, srow)

    xt = jnp.concatenate(
        [b, s[:, None], jnp.zeros((_NP, _W - 5), jnp.float32)], axis=1)
    xst = _get_sc_permute()(xt, rank.reshape(_NP))

    shp = jax.ShapeDtypeStruct((1, _NP), jnp.float32)
    o1, o2, o3, o4, o5 = pl.pallas_call(
        _nms_body,
        out_shape=[shp] * 5,
        scratch_shapes=[pltpu.VMEM((_B, _B), jnp.float32)],
    )(xst)

    out = jnp.stack([o1.reshape(_NP), o2.reshape(_NP), o3.reshape(_NP),
                     o4.reshape(_NP), o5.reshape(_NP)], axis=1)
    return out[:n]


# speedup vs baseline: 104.6735x; 2.6487x over previous
import jax
import jax.numpy as jnp
from jax import lax
from jax.experimental import pallas as pl
from jax.experimental.pallas import tpu as pltpu
from jax.experimental.pallas import tpu_sc as plsc
import functools as _functools

_N = 5000
_NP = 5120
_ROWS = 8
_COLS = _NP // _ROWS
_NMS_T = 0.3
_SCORE_T = 0.05

_NC = 2
_NS = 16
_NW = _NC * _NS
_RPW = _NP // _NW
_HALF = _RPW // 2
_W = 16


def _rank_body(scol_ref, srow_ref, rank_ref):
    f32 = jnp.float32
    srow = srow_ref[...]
    gidx_row = lax.broadcasted_iota(jnp.int32, (1, _NP), 1).astype(f32)
    chunk = 640
    for q in range(_NP // chunk):
        sc = scol_ref[q * chunk:(q + 1) * chunk, :]
        gc = lax.broadcasted_iota(jnp.int32, (chunk, 1), 0).astype(f32) \
            + float(q * chunk)
        cmp = (srow > sc) | ((srow == sc) & (gidx_row < gc))
        rank_ref[q * chunk:(q + 1) * chunk, :] = jnp.sum(
            cmp.astype(f32), axis=1, keepdims=True).astype(jnp.int32)


def _sc_permute_body(xt_hbm, rank_hbm, out_hbm, idx_a, idx_b, rows_a, rows_b,
                     sem):
    wid = lax.axis_index("s") * _NC + lax.axis_index("c")
    base = wid * _RPW
    pltpu.sync_copy(rank_hbm.at[pl.ds(base, _HALF)], idx_a)
    pltpu.sync_copy(rank_hbm.at[pl.ds(base + _HALF, _HALF)], idx_b)
    pltpu.sync_copy(xt_hbm.at[pl.ds(base, _HALF), :], rows_a)
    pltpu.sync_copy(xt_hbm.at[pl.ds(base + _HALF, _HALF), :], rows_b)
    pltpu.async_copy(rows_a, out_hbm.at[idx_a], sem).wait()
    pltpu.async_copy(rows_b, out_hbm.at[idx_b], sem).wait()


@_functools.lru_cache(maxsize=1)
def _get_sc_permute():
    return pl.kernel(
        _sc_permute_body,
        out_type=jax.ShapeDtypeStruct((_NP, _W), jnp.float32),
        mesh=plsc.VectorSubcoreMesh(core_axis_name="c", subcore_axis_name="s"),
        scratch_types=[
            pltpu.VMEM((_HALF,), jnp.int32),
            pltpu.VMEM((_HALF,), jnp.int32),
            pltpu.VMEM((_HALF, _W), jnp.float32),
            pltpu.VMEM((_HALF, _W), jnp.float32),
            pltpu.SemaphoreType.DMA,
        ],
        compiler_params=pltpu.CompilerParams(use_tc_tiling_on_sc=False),
    )


_B = 128


def _nms_body(xst_ref, o1, o2, o3, o4, o5):
    f32 = jnp.float32
    bf16 = jnp.bfloat16
    parts = [[] for _ in range(5)]
    for q in range(_ROWS):
        t = jnp.transpose(xst_ref[q * _COLS:(q + 1) * _COLS, :])
        for c in range(5):
            parts[c].append(t[c:c + 1, :])
    x1 = jnp.concatenate(parts[0], axis=1)
    y1 = jnp.concatenate(parts[1], axis=1)
    x2 = jnp.concatenate(parts[2], axis=1)
    y2 = jnp.concatenate(parts[3], axis=1)
    ss = jnp.concatenate(parts[4], axis=1)
    area = (x2 - x1) * (y2 - y1)
    v = xst_ref[:, 0:5]
    hi = v.astype(bf16).astype(f32)
    r = v - hi
    lo = r.astype(bf16).astype(f32)
    lolo = r - lo
    xsplit16 = jnp.concatenate(
        [hi, lo, lolo, jnp.zeros((_NP, 1), f32)], axis=1).astype(bf16)

    biota_c = lax.broadcasted_iota(jnp.int32, (_B, 1), 0).astype(f32)
    laneiota = lax.broadcasted_iota(jnp.int32, (1, _B), 1)
    tri = (lax.broadcasted_iota(jnp.int32, (_B, _B), 0)
           < lax.broadcasted_iota(jnp.int32, (_B, _B), 1))

    alive0 = (ss > _SCORE_T).astype(f32)
    keep0 = jnp.zeros((1, _NP), f32)
    cnt0 = jnp.sum(alive0)

    def cond(state):
        return state[2] > 0.0

    def body(state):
        alive, keep, _ = state
        cum = alive
        k = 1
        while k < _NP:
            cum = cum + jnp.pad(cum[:, :-k], ((0, 0), (k, 0)))
            k *= 2
        excl = cum - alive
        self32 = jnp.where((alive > 0.0) & (excl < float(_B)), 1.0, 0.0)
        C = ((excl == biota_c) & (alive > 0.0)).astype(f32)
        C16 = C.astype(bf16)
        blk = jax.lax.dot_general(
            C16, xsplit16, (((1,), (0,)), ((), ())),
            preferred_element_type=f32)
        blkT = jnp.transpose(blk)
        bx1c = (blk[:, 0:1] + blk[:, 5:6]) + blk[:, 10:11]
        by1c = (blk[:, 1:2] + blk[:, 6:7]) + blk[:, 11:12]
        bx2c = (blk[:, 2:3] + blk[:, 7:8]) + blk[:, 12:13]
        by2c = (blk[:, 3:4] + blk[:, 8:9]) + blk[:, 13:14]
        bx1r = (blkT[0:1, :] + blkT[5:6, :]) + blkT[10:11, :]
        by1r = (blkT[1:2, :] + blkT[6:7, :]) + blkT[11:12, :]
        bx2r = (blkT[2:3, :] + blkT[7:8, :]) + blkT[12:13, :]
        by2r = (blkT[3:4, :] + blkT[8:9, :]) + blkT[13:14, :]
        areac = (bx2c - bx1c) * (by2c - by1c)
        arear = (bx2r - bx1r) * (by2r - by1r)
        wb = jnp.maximum(jnp.minimum(bx2c, bx2r) - jnp.maximum(bx1c, bx1r), 0.0)
        hb = jnp.maximum(jnp.minimum(by2c, by2r) - jnp.maximum(by1c, by1r), 0.0)
        interb = wb * hb
        ioub = interb / jnp.maximum((areac + arear) - interb, 1e-9)
        S16 = jnp.where((ioub > _NMS_T) & tri, 1.0, 0.0).astype(bf16)

        nm = jnp.sum(self32)
        valid = (laneiota.astype(f32) < nm).astype(f32)

        def jcond(st):
            return st[1]

        def jbody(st):
            a, _ = st
            deadb = jax.lax.dot_general(
                a.astype(bf16), S16, (((1,), (0,)), ((), ())),
                preferred_element_type=f32)
            anew = jnp.where(deadb > 0.5, 0.0, valid)
            return anew, jnp.sum(jnp.abs(anew - a)) > 0.0

        a, _ = jax.lax.while_loop(jcond, jbody, (valid, jnp.bool_(True)))

        w = jnp.maximum(jnp.minimum(x2, bx2c) - jnp.maximum(x1, bx1c), 0.0)
        h = jnp.maximum(jnp.minimum(y2, by2c) - jnp.maximum(y1, by1c), 0.0)
        inter = w * h
        iou = inter / jnp.maximum((areac + area) - inter, 1e-9)
        P16 = (iou > _NMS_T).astype(bf16)
        a16 = a.astype(bf16)
        deadcnt = jax.lax.dot_general(
            a16, P16, (((1,), (0,)), ((), ())), preferred_element_type=f32)
        keep = jnp.maximum(keep, jax.lax.dot_general(
            a16, C16, (((1,), (0,)), ((), ())), preferred_element_type=f32))
        alive = jnp.where(deadcnt > 0.5, 0.0, alive * (1.0 - self32))
        return alive, keep, jnp.sum(alive)

    _, keep, _ = jax.lax.while_loop(cond, body, (alive0, keep0, cnt0))

    o1[...] = x1 * keep
    o2[...] = y1 * keep
    o3[...] = x2 * keep
    o4[...] = y2 * keep
    o5[...] = ss * keep


@jax.jit
def kernel(boxes, scores):
    n = boxes.shape[0]
    pad = _NP - n
    b = jnp.pad(boxes, ((0, pad), (0, 0)))
    s = jnp.pad(scores, ((0, pad),), constant_values=-1.0)
    scol = s.reshape(_NP, 1)
    srow = s.reshape(1, _NP)

    rank = pl.pallas_call(
        _rank_body,
        out_shape=jax.ShapeDtypeStruct((_NP, 1), jnp.int32),
    )(scol, srow)

    xt = jnp.concatenate(
        [b, s[:, None], jnp.zeros((_NP, _W - 5), jnp.float32)], axis=1)
    xst = _get_sc_permute()(xt, rank.reshape(_NP))

    shp = jax.ShapeDtypeStruct((1, _NP), jnp.float32)
    o1, o2, o3, o4, o5 = pl.pallas_call(
        _nms_body,
        out_shape=[shp] * 5,
    )(xst)

    out = jnp.stack([o1.reshape(_NP), o2.reshape(_NP), o3.reshape(_NP),
                     o4.reshape(_NP), o5.reshape(_NP)], axis=1)
    return out[:n]
